# baseline (speedup 1.0000x reference)
import functools

import jax
import jax.numpy as jnp
from jax import lax
from jax.experimental import pallas as pl
from jax.experimental.pallas import tpu as pltpu
from jax.experimental.pallas import tpu_sc as plsc

N = 10000
E = 160000
CW = 128
NSUB = 16
NCORE = 2
ROWS_PER_SUB = N // NSUB

B = 40
NB = (E // NSUB) // B
NBUF = 4

ZR = 40

BD = 40
NBD = (E // (NSUB * NCORE)) // BD
DW = 16

_MESH = plsc.VectorSubcoreMesh(core_axis_name="c", subcore_axis_name="s")
_SC_PARAMS = pltpu.CompilerParams(use_tc_tiling_on_sc=False)


def _make_segsum(C):
    cpc = C // NCORE

    @functools.partial(
        pl.kernel,
        out_type=jax.ShapeDtypeStruct((C, N, CW), jnp.float32),
        mesh=_MESH,
        scratch_types=[
            pltpu.VMEM((NB, B), jnp.int32),
            pltpu.VMEM((NB, B), jnp.int32),
            pltpu.VMEM((NBUF, B, CW), jnp.float32),
            pltpu.VMEM((ZR, CW), jnp.float32),
            pltpu.VMEM_SHARED((N, CW), jnp.float32),
            pltpu.SemaphoreType.DMA,
            pltpu.SemaphoreType.DMA,
        ],
        compiler_params=_SC_PARAMS,
    )
    def segsum(x_hbm, srcw_hbm, dstw_hbm, zer_hbm, out_hbm,
               src_v, dst_v, buf, z_v, acc, gsem, ssem):
        core = lax.axis_index("c")
        sub = lax.axis_index("s")
        row0 = sub * ROWS_PER_SUB
        pltpu.sync_copy(srcw_hbm.at[sub], src_v)
        pltpu.sync_copy(dstw_hbm.at[sub], dst_v)
        pltpu.sync_copy(zer_hbm, z_v)
        for ci in range(cpc):
            c = core * cpc + ci
            xc = x_hbm.at[c]
            for r in range(ROWS_PER_SUB // ZR):
                pltpu.sync_copy(z_v, acc.at[pl.ds(row0 + r * ZR, ZR)])
            if ROWS_PER_SUB % ZR:
                pltpu.sync_copy(z_v.at[pl.ds(0, ROWS_PER_SUB % ZR)],
                                acc.at[pl.ds(row0 + (ROWS_PER_SUB // ZR) * ZR,
                                             ROWS_PER_SUB % ZR)])
            plsc.subcore_barrier()
            pltpu.async_copy(xc.at[src_v.at[0]], buf.at[0], gsem)
            pltpu.async_copy(xc.at[src_v.at[1]], buf.at[1], gsem)

            @pl.loop(0, NB)
            def _batch(j):
                @pl.when(j >= 2)
                def _():
                    pltpu.make_async_copy(buf.at[(j - 2) % NBUF],
                                          acc.at[dst_v.at[j - 2]], ssem).wait()

                @pl.when(j + 2 < NB)
                def _():
                    pltpu.async_copy(xc.at[src_v.at[j + 2]],
                                     buf.at[(j + 2) % NBUF], gsem)

                pltpu.make_async_copy(xc.at[src_v.at[j]], buf.at[j % NBUF],
                                      gsem).wait()
                pltpu.async_copy(buf.at[j % NBUF], acc.at[dst_v.at[j]], ssem,
                                 add=True)

            for j in (NB - 2, NB - 1):
                pltpu.make_async_copy(buf.at[j % NBUF], acc.at[dst_v.at[j]],
                                      ssem).wait()
            plsc.subcore_barrier()
            pltpu.sync_copy(acc.at[pl.ds(row0, ROWS_PER_SUB)],
                            out_hbm.at[c].at[pl.ds(row0, ROWS_PER_SUB)])
    return segsum


@functools.partial(
    pl.kernel,
    out_type=jax.ShapeDtypeStruct((NCORE, N, DW), jnp.float32),
    mesh=_MESH,
    scratch_types=[
        pltpu.VMEM((NBD, BD), jnp.int32),
        pltpu.VMEM((BD, DW), jnp.float32),
        pltpu.VMEM((ROWS_PER_SUB, DW), jnp.float32),
        pltpu.VMEM_SHARED((N, DW), jnp.float32),
    ],
    compiler_params=_SC_PARAMS,
)
def _deg_kernel(dstw_hbm, ones_hbm, zer_hbm, out_hbm, dst_v, ones_v, z_v, acc):
    core = lax.axis_index("c")
    sub = lax.axis_index("s")
    w = core * NSUB + sub
    row0 = sub * ROWS_PER_SUB
    pltpu.sync_copy(dstw_hbm.at[w], dst_v)
    pltpu.sync_copy(ones_hbm, ones_v)
    pltpu.sync_copy(zer_hbm, z_v)
    pltpu.sync_copy(z_v, acc.at[pl.ds(row0, ROWS_PER_SUB)])
    plsc.subcore_barrier()

    @pl.loop(0, NBD)
    def _batch(j):
        pltpu.sync_copy(ones_v, acc.at[dst_v.at[j]], add=True)

    plsc.subcore_barrier()
    pltpu.sync_copy(acc.at[pl.ds(row0, ROWS_PER_SUB)],
                    out_hbm.at[core].at[pl.ds(row0, ROWS_PER_SUB)])


def _tc_layer(aggc, deg2, W, b, g, be, rows=1000):
    cin = aggc.shape[0]
    hout = W.shape[1]
    cout = hout // CW
    nb = N // rows

    def body(agg_ref, deg_ref, w_ref, b_ref, g_ref, be_ref, out_ref):
        acc = jnp.zeros((rows, hout), jnp.float32)
        for c in range(cin):
            acc += jnp.dot(agg_ref[c].astype(jnp.bfloat16),
                           w_ref[c * CW:(c + 1) * CW, :],
                           preferred_element_type=jnp.float32)
        deg = deg_ref[0, :, 0:1] + deg_ref[1, :, 0:1]
        scale = 1.0 / jnp.maximum(deg, 1.0)
        acc = acc * scale + b_ref[0]
        mu = jnp.mean(acc, axis=1, keepdims=True)
        xm = acc - mu
        var = jnp.mean(xm * xm, axis=1, keepdims=True)
        y = xm * lax.rsqrt(var + 1e-5) * g_ref[0] + be_ref[0]
        y = jnp.maximum(y, 0.0)
        for c in range(cout):
            out_ref[c] = y[:, c * CW:(c + 1) * CW]

    return pl.pallas_call(
        body,
        grid=(nb,),
        in_specs=[
            pl.BlockSpec((cin, rows, CW), lambda i: (0, i, 0)),
            pl.BlockSpec((NCORE, rows, DW), lambda i: (0, i, 0)),
            pl.BlockSpec((cin * CW, hout), lambda i: (0, 0)),
            pl.BlockSpec((1, hout), lambda i: (0, 0)),
            pl.BlockSpec((1, hout), lambda i: (0, 0)),
            pl.BlockSpec((1, hout), lambda i: (0, 0)),
        ],
        out_specs=pl.BlockSpec((cout, rows, CW), lambda i: (0, i, 0)),
        out_shape=jax.ShapeDtypeStruct((cout, N, CW), jnp.float32),
    )(aggc, deg2, W, b.reshape(1, hout), g.reshape(1, hout), be.reshape(1, hout))


def _tc_final(hc, lW1, lb1, g1, be1, lW2, lb2, rows=1000):
    cin = hc.shape[0]
    h = lW1.shape[1]
    dout = lW2.shape[1]
    nb = N // rows

    def body(in_ref, w1_ref, b1_ref, g_ref, be_ref, w2_ref, b2_ref, out_ref):
        acc = jnp.zeros((rows, h), jnp.float32)
        for c in range(cin):
            acc += jnp.dot(in_ref[c].astype(jnp.bfloat16),
                           w1_ref[c * CW:(c + 1) * CW, :],
                           preferred_element_type=jnp.float32)
        acc = acc + b1_ref[0]
        mu = jnp.mean(acc, axis=1, keepdims=True)
        xm = acc - mu
        var = jnp.mean(xm * xm, axis=1, keepdims=True)
        y = xm * lax.rsqrt(var + 1e-5) * g_ref[0] + be_ref[0]
        y = jnp.maximum(y, 0.0)
        out_ref[...] = jnp.dot(y.astype(jnp.bfloat16), w2_ref[...],
                               preferred_element_type=jnp.float32) + b2_ref[0]

    return pl.pallas_call(
        body,
        grid=(nb,),
        in_specs=[
            pl.BlockSpec((cin, rows, CW), lambda i: (0, i, 0)),
            pl.BlockSpec((cin * CW, h), lambda i: (0, 0)),
            pl.BlockSpec((1, h), lambda i: (0, 0)),
            pl.BlockSpec((1, h), lambda i: (0, 0)),
            pl.BlockSpec((1, h), lambda i: (0, 0)),
            pl.BlockSpec((h, dout), lambda i: (0, 0)),
            pl.BlockSpec((1, dout), lambda i: (0, 0)),
        ],
        out_specs=pl.BlockSpec((rows, dout), lambda i: (i, 0)),
        out_shape=jax.ShapeDtypeStruct((N, dout), jnp.float32),
    )(hc, lW1, lb1.reshape(1, h), g1.reshape(1, h), be1.reshape(1, h),
      lW2, lb2.reshape(1, dout))


def kernel(feat, edge_index, W1, b1, W2, b2, W3, b3, W4, b4, W5, b5,
           lW1, lb1, lW2, lb2, g1, be1, g2, be2, g3, be3, g4, be4, g5, be5):
    src = edge_index[0]
    dst = edge_index[1]
    srcw = src.reshape(NSUB, NB, B)
    dstw = dst.reshape(NSUB, NB, B)
    dstd = dst.reshape(NSUB * NCORE, NBD, BD)
    bf = jnp.bfloat16
    W1, W2, W3, W4, W5 = (W.astype(bf) for W in (W1, W2, W3, W4, W5))
    lW1b, lW2b = lW1.astype(bf), lW2.astype(bf)

    ones_d = jnp.ones((BD, DW), jnp.float32)
    zer_d = jnp.zeros((ROWS_PER_SUB, DW), jnp.float32)
    zer_f = jnp.zeros((ZR, CW), jnp.float32)

    deg2 = _deg_kernel(dstd, ones_d, zer_d)

    c0 = feat.shape[1] // CW
    hc = feat.reshape(N, c0, CW).transpose(1, 0, 2)

    segsum_in = _make_segsum(c0)
    segsum_h = _make_segsum(2048 // CW)

    agg = segsum_in(hc, srcw, dstw, zer_f)
    hc = _tc_layer(agg, deg2, W1, b1, g1, be1)
    for (W, b, g, be) in ((W2, b2, g2, be2), (W3, b3, g3, be3),
                          (W4, b4, g4, be4), (W5, b5, g5, be5)):
        agg = segsum_h(hc, srcw, dstw, zer_f)
        hc = _tc_layer(agg, deg2, W, b, g, be)
    return _tc_final(hc, lW1b, lb1, g1, be1, lW2b, lb2)


# speedup vs baseline: 1.0119x; 1.0119x over previous
import functools

import jax
import jax.numpy as jnp
from jax import lax
from jax.experimental import pallas as pl
from jax.experimental.pallas import tpu as pltpu
from jax.experimental.pallas import tpu_sc as plsc

N = 10000
E = 160000
CW = 128
NSUB = 16
NCORE = 2
ROWS_PER_SUB = N // NSUB

B = 40
NB = (E // NSUB) // B
NBUF = 4

ZR = 40

BD = 40
NBD = (E // (NSUB * NCORE)) // BD
DW = 16

_MESH = plsc.VectorSubcoreMesh(core_axis_name="c", subcore_axis_name="s")
_SC_PARAMS = pltpu.CompilerParams(use_tc_tiling_on_sc=False)


def _make_segsum(C):
    cpc = C // NCORE

    @functools.partial(
        pl.kernel,
        out_type=jax.ShapeDtypeStruct((C, N, CW), jnp.float32),
        mesh=_MESH,
        scratch_types=[
            pltpu.VMEM((NB, B), jnp.int32),
            pltpu.VMEM((NB, B), jnp.int32),
            pltpu.VMEM((NBUF, B, CW), jnp.float32),
            pltpu.VMEM((ZR, CW), jnp.float32),
            pltpu.VMEM_SHARED((N, CW), jnp.float32),
            pltpu.SemaphoreType.DMA,
            pltpu.SemaphoreType.DMA,
        ],
        compiler_params=_SC_PARAMS,
    )
    def segsum(x_hbm, srcw_hbm, dstw_hbm, zer_hbm, out_hbm,
               src_v, dst_v, buf, z_v, acc, gsem, ssem):
        core = lax.axis_index("c")
        sub = lax.axis_index("s")
        row0 = sub * ROWS_PER_SUB
        pltpu.sync_copy(srcw_hbm.at[sub], src_v)
        pltpu.sync_copy(dstw_hbm.at[sub], dst_v)
        pltpu.sync_copy(zer_hbm, z_v)
        for ci in range(cpc):
            c = core * cpc + ci
            xc = x_hbm.at[c]
            for r in range(ROWS_PER_SUB // ZR):
                pltpu.sync_copy(z_v, acc.at[pl.ds(row0 + r * ZR, ZR)])
            if ROWS_PER_SUB % ZR:
                pltpu.sync_copy(z_v.at[pl.ds(0, ROWS_PER_SUB % ZR)],
                                acc.at[pl.ds(row0 + (ROWS_PER_SUB // ZR) * ZR,
                                             ROWS_PER_SUB % ZR)])
            plsc.subcore_barrier()
            pltpu.async_copy(xc.at[src_v.at[0]], buf.at[0], gsem)
            pltpu.async_copy(xc.at[src_v.at[1]], buf.at[1], gsem)

            @pl.loop(0, NB)
            def _batch(j):
                @pl.when(j >= 2)
                def _():
                    pltpu.make_async_copy(buf.at[(j - 2) % NBUF],
                                          acc.at[dst_v.at[j - 2]], ssem).wait()

                @pl.when(j + 2 < NB)
                def _():
                    pltpu.async_copy(xc.at[src_v.at[j + 2]],
                                     buf.at[(j + 2) % NBUF], gsem)

                pltpu.make_async_copy(xc.at[src_v.at[j]], buf.at[j % NBUF],
                                      gsem).wait()
                pltpu.async_copy(buf.at[j % NBUF], acc.at[dst_v.at[j]], ssem,
                                 add=True)

            for j in (NB - 2, NB - 1):
                pltpu.make_async_copy(buf.at[j % NBUF], acc.at[dst_v.at[j]],
                                      ssem).wait()
            plsc.subcore_barrier()
            pltpu.sync_copy(acc.at[pl.ds(row0, ROWS_PER_SUB)],
                            out_hbm.at[c].at[pl.ds(row0, ROWS_PER_SUB)])
    return segsum


@functools.partial(
    pl.kernel,
    out_type=jax.ShapeDtypeStruct((NCORE, N, DW), jnp.float32),
    mesh=_MESH,
    scratch_types=[
        pltpu.VMEM((NBD, BD), jnp.int32),
        pltpu.VMEM((BD, DW), jnp.float32),
        pltpu.VMEM((ROWS_PER_SUB, DW), jnp.float32),
        pltpu.VMEM_SHARED((N, DW), jnp.float32),
    ],
    compiler_params=_SC_PARAMS,
)
def _deg_kernel(dstw_hbm, ones_hbm, zer_hbm, out_hbm, dst_v, ones_v, z_v, acc):
    core = lax.axis_index("c")
    sub = lax.axis_index("s")
    w = core * NSUB + sub
    row0 = sub * ROWS_PER_SUB
    pltpu.sync_copy(dstw_hbm.at[w], dst_v)
    pltpu.sync_copy(ones_hbm, ones_v)
    pltpu.sync_copy(zer_hbm, z_v)
    pltpu.sync_copy(z_v, acc.at[pl.ds(row0, ROWS_PER_SUB)])
    plsc.subcore_barrier()

    @pl.loop(0, NBD)
    def _batch(j):
        pltpu.sync_copy(ones_v, acc.at[dst_v.at[j]], add=True)

    plsc.subcore_barrier()
    pltpu.sync_copy(acc.at[pl.ds(row0, ROWS_PER_SUB)],
                    out_hbm.at[core].at[pl.ds(row0, ROWS_PER_SUB)])


def _tc_layer(aggc, deg2, W, b, g, be, rows=400):
    cin = aggc.shape[0]
    hout = W.shape[1]
    cout = hout // CW
    nb = N // rows

    def body(agg_ref, deg_ref, w_ref, b_ref, g_ref, be_ref, out_ref, wbf_ref):
        @pl.when(pl.program_id(0) == 0)
        def _():
            wbf_ref[...] = w_ref[...].astype(jnp.bfloat16)

        acc = jnp.zeros((rows, hout), jnp.float32)
        for c in range(cin):
            acc += jnp.dot(agg_ref[c].astype(jnp.bfloat16),
                           wbf_ref[c * CW:(c + 1) * CW, :],
                           preferred_element_type=jnp.float32)
        deg = deg_ref[0, :, 0:1] + deg_ref[1, :, 0:1]
        scale = 1.0 / jnp.maximum(deg, 1.0)
        acc = acc * scale + b_ref[0]
        mu = jnp.mean(acc, axis=1, keepdims=True)
        xm = acc - mu
        var = jnp.mean(xm * xm, axis=1, keepdims=True)
        y = xm * lax.rsqrt(var + 1e-5) * g_ref[0] + be_ref[0]
        y = jnp.maximum(y, 0.0)
        for c in range(cout):
            out_ref[c] = y[:, c * CW:(c + 1) * CW]

    return pl.pallas_call(
        body,
        grid=(nb,),
        in_specs=[
            pl.BlockSpec((cin, rows, CW), lambda i: (0, i, 0)),
            pl.BlockSpec((NCORE, rows, DW), lambda i: (0, i, 0)),
            pl.BlockSpec((cin * CW, hout), lambda i: (0, 0)),
            pl.BlockSpec((1, hout), lambda i: (0, 0)),
            pl.BlockSpec((1, hout), lambda i: (0, 0)),
            pl.BlockSpec((1, hout), lambda i: (0, 0)),
        ],
        out_specs=pl.BlockSpec((cout, rows, CW), lambda i: (0, i, 0)),
        out_shape=jax.ShapeDtypeStruct((cout, N, CW), jnp.float32),
        scratch_shapes=[pltpu.VMEM((cin * CW, hout), jnp.bfloat16)],
    )(aggc, deg2, W, b.reshape(1, hout), g.reshape(1, hout), be.reshape(1, hout))


def _tc_final(hc, lW1, lb1, g1, be1, lW2, lb2, rows=400):
    cin = hc.shape[0]
    h = lW1.shape[1]
    dout = lW2.shape[1]
    nb = N // rows

    def body(in_ref, w1_ref, b1_ref, g_ref, be_ref, w2_ref, b2_ref, out_ref,
             wbf_ref, w2bf_ref):
        @pl.when(pl.program_id(0) == 0)
        def _():
            wbf_ref[...] = w1_ref[...].astype(jnp.bfloat16)
            w2bf_ref[...] = w2_ref[...].astype(jnp.bfloat16)

        acc = jnp.zeros((rows, h), jnp.float32)
        for c in range(cin):
            acc += jnp.dot(in_ref[c].astype(jnp.bfloat16),
                           wbf_ref[c * CW:(c + 1) * CW, :],
                           preferred_element_type=jnp.float32)
        acc = acc + b1_ref[0]
        mu = jnp.mean(acc, axis=1, keepdims=True)
        xm = acc - mu
        var = jnp.mean(xm * xm, axis=1, keepdims=True)
        y = xm * lax.rsqrt(var + 1e-5) * g_ref[0] + be_ref[0]
        y = jnp.maximum(y, 0.0)
        out_ref[...] = jnp.dot(y.astype(jnp.bfloat16), w2bf_ref[...],
                               preferred_element_type=jnp.float32) + b2_ref[0]

    return pl.pallas_call(
        body,
        grid=(nb,),
        in_specs=[
            pl.BlockSpec((cin, rows, CW), lambda i: (0, i, 0)),
            pl.BlockSpec((cin * CW, h), lambda i: (0, 0)),
            pl.BlockSpec((1, h), lambda i: (0, 0)),
            pl.BlockSpec((1, h), lambda i: (0, 0)),
            pl.BlockSpec((1, h), lambda i: (0, 0)),
            pl.BlockSpec((h, dout), lambda i: (0, 0)),
            pl.BlockSpec((1, dout), lambda i: (0, 0)),
        ],
        out_specs=pl.BlockSpec((rows, dout), lambda i: (i, 0)),
        out_shape=jax.ShapeDtypeStruct((N, dout), jnp.float32),
        scratch_shapes=[pltpu.VMEM((cin * CW, h), jnp.bfloat16),
                        pltpu.VMEM((h, dout), jnp.bfloat16)],
    )(hc, lW1, lb1.reshape(1, h), g1.reshape(1, h), be1.reshape(1, h),
      lW2, lb2.reshape(1, dout))


def kernel(feat, edge_index, W1, b1, W2, b2, W3, b3, W4, b4, W5, b5,
           lW1, lb1, lW2, lb2, g1, be1, g2, be2, g3, be3, g4, be4, g5, be5):
    src = edge_index[0]
    dst = edge_index[1]
    srcw = src.reshape(NSUB, NB, B)
    dstw = dst.reshape(NSUB, NB, B)
    dstd = dst.reshape(NSUB * NCORE, NBD, BD)

    ones_d = jnp.ones((BD, DW), jnp.float32)
    zer_d = jnp.zeros((ROWS_PER_SUB, DW), jnp.float32)
    zer_f = jnp.zeros((ZR, CW), jnp.float32)

    deg2 = _deg_kernel(dstd, ones_d, zer_d)

    c0 = feat.shape[1] // CW
    hc = feat.reshape(N, c0, CW).transpose(1, 0, 2)

    segsum_in = _make_segsum(c0)
    segsum_h = _make_segsum(2048 // CW)

    agg = segsum_in(hc, srcw, dstw, zer_f)
    hc = _tc_layer(agg, deg2, W1, b1, g1, be1)
    for (W, b, g, be) in ((W2, b2, g2, be2), (W3, b3, g3, be3),
                          (W4, b4, g4, be4), (W5, b5, g5, be5)):
        agg = segsum_h(hc, srcw, dstw, zer_f)
        hc = _tc_layer(agg, deg2, W, b, g, be)
    return _tc_final(hc, lW1, lb1, g1, be1, lW2, lb2)


# speedup vs baseline: 1.0121x; 1.0002x over previous
import functools

import jax
import jax.numpy as jnp
from jax import lax
from jax.experimental import pallas as pl
from jax.experimental.pallas import tpu as pltpu
from jax.experimental.pallas import tpu_sc as plsc

N = 10000
E = 160000
CW = 128
NSUB = 16
NCORE = 2
ROWS_PER_SUB = N // NSUB

B = 40
NB = (E // NSUB) // B
NBUF = 4

ZR = 40

BD = 40
NBD = (E // (NSUB * NCORE)) // BD
DW = 16

_MESH = plsc.VectorSubcoreMesh(core_axis_name="c", subcore_axis_name="s")
_SC_PARAMS = pltpu.CompilerParams(use_tc_tiling_on_sc=False)


def _make_segsum(C):
    cpc = C // NCORE

    @functools.partial(
        pl.kernel,
        out_type=jax.ShapeDtypeStruct((C, N, CW), jnp.float32),
        mesh=_MESH,
        scratch_types=[
            pltpu.VMEM((NB, B), jnp.int32),
            pltpu.VMEM((NB, B), jnp.int32),
            pltpu.VMEM((NBUF, B, CW), jnp.float32),
            pltpu.VMEM((ZR, CW), jnp.float32),
            pltpu.VMEM_SHARED((N, CW), jnp.float32),
            pltpu.SemaphoreType.DMA,
            pltpu.SemaphoreType.DMA,
        ],
        compiler_params=_SC_PARAMS,
    )
    def segsum(x_hbm, srcw_hbm, dstw_hbm, zer_hbm, out_hbm,
               src_v, dst_v, buf, z_v, acc, gsem, ssem):
        core = lax.axis_index("c")
        sub = lax.axis_index("s")
        row0 = sub * ROWS_PER_SUB
        pltpu.sync_copy(srcw_hbm.at[sub], src_v)
        pltpu.sync_copy(dstw_hbm.at[sub], dst_v)
        pltpu.sync_copy(zer_hbm, z_v)
        for ci in range(cpc):
            c = core * cpc + ci
            xc = x_hbm.at[c]
            for r in range(ROWS_PER_SUB // ZR):
                pltpu.sync_copy(z_v, acc.at[pl.ds(row0 + r * ZR, ZR)])
            if ROWS_PER_SUB % ZR:
                pltpu.sync_copy(z_v.at[pl.ds(0, ROWS_PER_SUB % ZR)],
                                acc.at[pl.ds(row0 + (ROWS_PER_SUB // ZR) * ZR,
                                             ROWS_PER_SUB % ZR)])
            plsc.subcore_barrier()
            pltpu.async_copy(xc.at[src_v.at[0]], buf.at[0], gsem)
            pltpu.async_copy(xc.at[src_v.at[1]], buf.at[1], gsem)

            @pl.loop(0, NB)
            def _batch(j):
                @pl.when(j >= 2)
                def _():
                    pltpu.make_async_copy(buf.at[(j - 2) % NBUF],
                                          acc.at[dst_v.at[j - 2]], ssem).wait()

                @pl.when(j + 2 < NB)
                def _():
                    pltpu.async_copy(xc.at[src_v.at[j + 2]],
                                     buf.at[(j + 2) % NBUF], gsem)

                pltpu.make_async_copy(xc.at[src_v.at[j]], buf.at[j % NBUF],
                                      gsem).wait()
                pltpu.async_copy(buf.at[j % NBUF], acc.at[dst_v.at[j]], ssem,
                                 add=True)

            for j in (NB - 2, NB - 1):
                pltpu.make_async_copy(buf.at[j % NBUF], acc.at[dst_v.at[j]],
                                      ssem).wait()
            plsc.subcore_barrier()
            pltpu.sync_copy(acc.at[pl.ds(row0, ROWS_PER_SUB)],
                            out_hbm.at[c].at[pl.ds(row0, ROWS_PER_SUB)])
    return segsum


@functools.partial(
    pl.kernel,
    out_type=jax.ShapeDtypeStruct((NCORE, N, DW), jnp.float32),
    mesh=_MESH,
    scratch_types=[
        pltpu.VMEM((NBD, BD), jnp.int32),
        pltpu.VMEM((BD, DW), jnp.float32),
        pltpu.VMEM((ROWS_PER_SUB, DW), jnp.float32),
        pltpu.VMEM_SHARED((N, DW), jnp.float32),
    ],
    compiler_params=_SC_PARAMS,
)
def _deg_kernel(dstw_hbm, ones_hbm, zer_hbm, out_hbm, dst_v, ones_v, z_v, acc):
    core = lax.axis_index("c")
    sub = lax.axis_index("s")
    w = core * NSUB + sub
    row0 = sub * ROWS_PER_SUB
    pltpu.sync_copy(dstw_hbm.at[w], dst_v)
    pltpu.sync_copy(ones_hbm, ones_v)
    pltpu.sync_copy(zer_hbm, z_v)
    pltpu.sync_copy(z_v, acc.at[pl.ds(row0, ROWS_PER_SUB)])
    plsc.subcore_barrier()

    @pl.loop(0, NBD)
    def _batch(j):
        pltpu.sync_copy(ones_v, acc.at[dst_v.at[j]], add=True)

    plsc.subcore_barrier()
    pltpu.sync_copy(acc.at[pl.ds(row0, ROWS_PER_SUB)],
                    out_hbm.at[core].at[pl.ds(row0, ROWS_PER_SUB)])


def _tc_layer(aggc, deg2, W, b, g, be, rows=400):
    cin = aggc.shape[0]
    hout = W.shape[1]
    cout = hout // CW
    nb = N // rows

    def body(agg_ref, deg_ref, w_ref, b_ref, g_ref, be_ref, out_ref):
        acc = jnp.zeros((rows, hout), jnp.float32)
        for c in range(cin):
            acc += jnp.dot(agg_ref[c].astype(jnp.bfloat16),
                           w_ref[c * CW:(c + 1) * CW, :],
                           preferred_element_type=jnp.float32)
        deg = deg_ref[0, :, 0:1] + deg_ref[1, :, 0:1]
        scale = 1.0 / jnp.maximum(deg, 1.0)
        acc = acc * scale + b_ref[0]
        mu = jnp.mean(acc, axis=1, keepdims=True)
        xm = acc - mu
        var = jnp.mean(xm * xm, axis=1, keepdims=True)
        y = xm * lax.rsqrt(var + 1e-5) * g_ref[0] + be_ref[0]
        y = jnp.maximum(y, 0.0)
        for c in range(cout):
            out_ref[c] = y[:, c * CW:(c + 1) * CW]

    return pl.pallas_call(
        body,
        grid=(nb,),
        in_specs=[
            pl.BlockSpec((cin, rows, CW), lambda i: (0, i, 0)),
            pl.BlockSpec((NCORE, rows, DW), lambda i: (0, i, 0)),
            pl.BlockSpec((cin * CW, hout), lambda i: (0, 0)),
            pl.BlockSpec((1, hout), lambda i: (0, 0)),
            pl.BlockSpec((1, hout), lambda i: (0, 0)),
            pl.BlockSpec((1, hout), lambda i: (0, 0)),
        ],
        out_specs=pl.BlockSpec((cout, rows, CW), lambda i: (0, i, 0)),
        out_shape=jax.ShapeDtypeStruct((cout, N, CW), jnp.float32),
    )(aggc, deg2, W, b.reshape(1, hout), g.reshape(1, hout), be.reshape(1, hout))


def _tc_final(hc, lW1, lb1, g1, be1, lW2, lb2, rows=400):
    cin = hc.shape[0]
    h = lW1.shape[1]
    dout = lW2.shape[1]
    nb = N // rows

    def body(in_ref, w1_ref, b1_ref, g_ref, be_ref, w2_ref, b2_ref, out_ref):
        acc = jnp.zeros((rows, h), jnp.float32)
        for c in range(cin):
            acc += jnp.dot(in_ref[c].astype(jnp.bfloat16),
                           w1_ref[c * CW:(c + 1) * CW, :],
                           preferred_element_type=jnp.float32)
        acc = acc + b1_ref[0]
        mu = jnp.mean(acc, axis=1, keepdims=True)
        xm = acc - mu
        var = jnp.mean(xm * xm, axis=1, keepdims=True)
        y = xm * lax.rsqrt(var + 1e-5) * g_ref[0] + be_ref[0]
        y = jnp.maximum(y, 0.0)
        out_ref[...] = jnp.dot(y.astype(jnp.bfloat16), w2_ref[...],
                               preferred_element_type=jnp.float32) + b2_ref[0]

    return pl.pallas_call(
        body,
        grid=(nb,),
        in_specs=[
            pl.BlockSpec((cin, rows, CW), lambda i: (0, i, 0)),
            pl.BlockSpec((cin * CW, h), lambda i: (0, 0)),
            pl.BlockSpec((1, h), lambda i: (0, 0)),
            pl.BlockSpec((1, h), lambda i: (0, 0)),
            pl.BlockSpec((1, h), lambda i: (0, 0)),
            pl.BlockSpec((h, dout), lambda i: (0, 0)),
            pl.BlockSpec((1, dout), lambda i: (0, 0)),
        ],
        out_specs=pl.BlockSpec((rows, dout), lambda i: (i, 0)),
        out_shape=jax.ShapeDtypeStruct((N, dout), jnp.float32),
    )(hc, lW1, lb1.reshape(1, h), g1.reshape(1, h), be1.reshape(1, h),
      lW2, lb2.reshape(1, dout))


def kernel(feat, edge_index, W1, b1, W2, b2, W3, b3, W4, b4, W5, b5,
           lW1, lb1, lW2, lb2, g1, be1, g2, be2, g3, be3, g4, be4, g5, be5):
    src = edge_index[0]
    dst = edge_index[1]
    srcw = src.reshape(NSUB, NB, B)
    dstw = dst.reshape(NSUB, NB, B)
    dstd = dst.reshape(NSUB * NCORE, NBD, BD)
    bf = jnp.bfloat16
    W1, W2, W3, W4, W5 = (W.astype(bf) for W in (W1, W2, W3, W4, W5))
    lW1, lW2 = lW1.astype(bf), lW2.astype(bf)

    ones_d = jnp.ones((BD, DW), jnp.float32)
    zer_d = jnp.zeros((ROWS_PER_SUB, DW), jnp.float32)
    zer_f = jnp.zeros((ZR, CW), jnp.float32)

    deg2 = _deg_kernel(dstd, ones_d, zer_d)

    c0 = feat.shape[1] // CW
    hc = feat.reshape(N, c0, CW).transpose(1, 0, 2)

    segsum_in = _make_segsum(c0)
    segsum_h = _make_segsum(2048 // CW)

    agg = segsum_in(hc, srcw, dstw, zer_f)
    hc = _tc_layer(agg, deg2, W1, b1, g1, be1)
    for (W, b, g, be) in ((W2, b2, g2, be2), (W3, b3, g3, be3),
                          (W4, b4, g4, be4), (W5, b5, g5, be5)):
        agg = segsum_h(hc, srcw, dstw, zer_f)
        hc = _tc_layer(agg, deg2, W, b, g, be)
    return _tc_final(hc, lW1, lb1, g1, be1, lW2, lb2)


# speedup vs baseline: 1.0929x; 1.0799x over previous
import functools

import jax
import jax.numpy as jnp
from jax import lax
from jax.experimental import pallas as pl
from jax.experimental.pallas import tpu as pltpu
from jax.experimental.pallas import tpu_sc as plsc

N = 10000
E = 160000
CW = 128
NSUB = 16
NCORE = 2
ROWS_PER_SUB = N // NSUB

B = 40
NB = (E // NSUB) // B
NBUF = 5

ZR = 40

BD = 40
NBD = (E // (NSUB * NCORE)) // BD
DW = 16

_MESH = plsc.VectorSubcoreMesh(core_axis_name="c", subcore_axis_name="s")
_SC_PARAMS = pltpu.CompilerParams(use_tc_tiling_on_sc=False)


def _make_segsum(C):
    cpc = C // NCORE

    @functools.partial(
        pl.kernel,
        out_type=jax.ShapeDtypeStruct((C, N, CW), jnp.float32),
        mesh=_MESH,
        scratch_types=[
            pltpu.VMEM((NB, B), jnp.int32),
            pltpu.VMEM((NB, B), jnp.int32),
            pltpu.VMEM((NBUF, B, CW), jnp.float32),
            pltpu.VMEM((ZR, CW), jnp.float32),
            pltpu.VMEM_SHARED((N, CW), jnp.float32),
            pltpu.SemaphoreType.DMA,
            pltpu.SemaphoreType.DMA,
        ],
        compiler_params=_SC_PARAMS,
    )
    def segsum(x_hbm, srcw_hbm, dstw_hbm, zer_hbm, out_hbm,
               src_v, dst_v, buf, z_v, acc, gsem, ssem):
        core = lax.axis_index("c")
        sub = lax.axis_index("s")
        row0 = sub * ROWS_PER_SUB
        pltpu.sync_copy(srcw_hbm.at[sub], src_v)
        pltpu.sync_copy(dstw_hbm.at[sub], dst_v)
        pltpu.sync_copy(zer_hbm, z_v)
        for ci in range(cpc):
            c = core * cpc + ci
            xc = x_hbm.at[c]
            for r in range(ROWS_PER_SUB // ZR):
                pltpu.sync_copy(z_v, acc.at[pl.ds(row0 + r * ZR, ZR)])
            if ROWS_PER_SUB % ZR:
                pltpu.sync_copy(z_v.at[pl.ds(0, ROWS_PER_SUB % ZR)],
                                acc.at[pl.ds(row0 + (ROWS_PER_SUB // ZR) * ZR,
                                             ROWS_PER_SUB % ZR)])
            plsc.subcore_barrier()
            pltpu.async_copy(xc.at[src_v.at[0]], buf.at[0], gsem)
            pltpu.async_copy(xc.at[src_v.at[1]], buf.at[1], gsem)
            pltpu.async_copy(xc.at[src_v.at[2]], buf.at[2], gsem)

            @pl.loop(0, NB)
            def _batch(j):
                @pl.when(j >= 2)
                def _():
                    pltpu.make_async_copy(buf.at[(j - 2) % NBUF],
                                          acc.at[dst_v.at[j - 2]], ssem).wait()

                @pl.when(j + 3 < NB)
                def _():
                    pltpu.async_copy(xc.at[src_v.at[j + 3]],
                                     buf.at[(j + 3) % NBUF], gsem)

                pltpu.make_async_copy(xc.at[src_v.at[j]], buf.at[j % NBUF],
                                      gsem).wait()
                pltpu.async_copy(buf.at[j % NBUF], acc.at[dst_v.at[j]], ssem,
                                 add=True)

            for j in (NB - 2, NB - 1):
                pltpu.make_async_copy(buf.at[j % NBUF], acc.at[dst_v.at[j]],
                                      ssem).wait()
            plsc.subcore_barrier()
            pltpu.sync_copy(acc.at[pl.ds(row0, ROWS_PER_SUB)],
                            out_hbm.at[c].at[pl.ds(row0, ROWS_PER_SUB)])
    return segsum


@functools.partial(
    pl.kernel,
    out_type=jax.ShapeDtypeStruct((NCORE, N, DW), jnp.float32),
    mesh=_MESH,
    scratch_types=[
        pltpu.VMEM((NBD, BD), jnp.int32),
        pltpu.VMEM((BD, DW), jnp.float32),
        pltpu.VMEM((ROWS_PER_SUB, DW), jnp.float32),
        pltpu.VMEM_SHARED((N, DW), jnp.float32),
    ],
    compiler_params=_SC_PARAMS,
)
def _deg_kernel(dstw_hbm, ones_hbm, zer_hbm, out_hbm, dst_v, ones_v, z_v, acc):
    core = lax.axis_index("c")
    sub = lax.axis_index("s")
    w = core * NSUB + sub
    row0 = sub * ROWS_PER_SUB
    pltpu.sync_copy(dstw_hbm.at[w], dst_v)
    pltpu.sync_copy(ones_hbm, ones_v)
    pltpu.sync_copy(zer_hbm, z_v)
    pltpu.sync_copy(z_v, acc.at[pl.ds(row0, ROWS_PER_SUB)])
    plsc.subcore_barrier()

    @pl.loop(0, NBD)
    def _batch(j):
        pltpu.sync_copy(ones_v, acc.at[dst_v.at[j]], add=True)

    plsc.subcore_barrier()
    pltpu.sync_copy(acc.at[pl.ds(row0, ROWS_PER_SUB)],
                    out_hbm.at[core].at[pl.ds(row0, ROWS_PER_SUB)])


def _tc_layer(aggc, deg2, W, b, g, be, rows=400):
    cin = aggc.shape[0]
    hout = W.shape[1]
    cout = hout // CW
    nb = N // rows

    def body(agg_ref, deg_ref, w_ref, b_ref, g_ref, be_ref, out_ref):
        acc = jnp.zeros((rows, hout), jnp.float32)
        for c in range(cin):
            acc += jnp.dot(agg_ref[c].astype(jnp.bfloat16),
                           w_ref[c * CW:(c + 1) * CW, :],
                           preferred_element_type=jnp.float32)
        deg = deg_ref[0, :, 0:1] + deg_ref[1, :, 0:1]
        scale = 1.0 / jnp.maximum(deg, 1.0)
        acc = acc * scale + b_ref[0]
        mu = jnp.mean(acc, axis=1, keepdims=True)
        xm = acc - mu
        var = jnp.mean(xm * xm, axis=1, keepdims=True)
        y = xm * lax.rsqrt(var + 1e-5) * g_ref[0] + be_ref[0]
        y = jnp.maximum(y, 0.0)
        for c in range(cout):
            out_ref[c] = y[:, c * CW:(c + 1) * CW]

    return pl.pallas_call(
        body,
        grid=(nb,),
        in_specs=[
            pl.BlockSpec((cin, rows, CW), lambda i: (0, i, 0)),
            pl.BlockSpec((NCORE, rows, DW), lambda i: (0, i, 0)),
            pl.BlockSpec((cin * CW, hout), lambda i: (0, 0)),
            pl.BlockSpec((1, hout), lambda i: (0, 0)),
            pl.BlockSpec((1, hout), lambda i: (0, 0)),
            pl.BlockSpec((1, hout), lambda i: (0, 0)),
        ],
        out_specs=pl.BlockSpec((cout, rows, CW), lambda i: (0, i, 0)),
        out_shape=jax.ShapeDtypeStruct((cout, N, CW), jnp.float32),
    )(aggc, deg2, W, b.reshape(1, hout), g.reshape(1, hout), be.reshape(1, hout))


def _tc_final(hc, lW1, lb1, g1, be1, lW2, lb2, rows=400):
    cin = hc.shape[0]
    h = lW1.shape[1]
    dout = lW2.shape[1]
    nb = N // rows

    def body(in_ref, w1_ref, b1_ref, g_ref, be_ref, w2_ref, b2_ref, out_ref):
        acc = jnp.zeros((rows, h), jnp.float32)
        for c in range(cin):
            acc += jnp.dot(in_ref[c].astype(jnp.bfloat16),
                           w1_ref[c * CW:(c + 1) * CW, :],
                           preferred_element_type=jnp.float32)
        acc = acc + b1_ref[0]
        mu = jnp.mean(acc, axis=1, keepdims=True)
        xm = acc - mu
        var = jnp.mean(xm * xm, axis=1, keepdims=True)
        y = xm * lax.rsqrt(var + 1e-5) * g_ref[0] + be_ref[0]
        y = jnp.maximum(y, 0.0)
        out_ref[...] = jnp.dot(y.astype(jnp.bfloat16), w2_ref[...],
                               preferred_element_type=jnp.float32) + b2_ref[0]

    return pl.pallas_call(
        body,
        grid=(nb,),
        in_specs=[
            pl.BlockSpec((cin, rows, CW), lambda i: (0, i, 0)),
            pl.BlockSpec((cin * CW, h), lambda i: (0, 0)),
            pl.BlockSpec((1, h), lambda i: (0, 0)),
            pl.BlockSpec((1, h), lambda i: (0, 0)),
            pl.BlockSpec((1, h), lambda i: (0, 0)),
            pl.BlockSpec((h, dout), lambda i: (0, 0)),
            pl.BlockSpec((1, dout), lambda i: (0, 0)),
        ],
        out_specs=pl.BlockSpec((rows, dout), lambda i: (i, 0)),
        out_shape=jax.ShapeDtypeStruct((N, dout), jnp.float32),
    )(hc, lW1, lb1.reshape(1, h), g1.reshape(1, h), be1.reshape(1, h),
      lW2, lb2.reshape(1, dout))


def kernel(feat, edge_index, W1, b1, W2, b2, W3, b3, W4, b4, W5, b5,
           lW1, lb1, lW2, lb2, g1, be1, g2, be2, g3, be3, g4, be4, g5, be5):
    src = edge_index[0]
    dst = edge_index[1]
    srcw = src.reshape(NSUB, NB, B)
    dstw = dst.reshape(NSUB, NB, B)
    dstd = dst.reshape(NSUB * NCORE, NBD, BD)
    bf = jnp.bfloat16
    W1, W2, W3, W4, W5 = (W.astype(bf) for W in (W1, W2, W3, W4, W5))
    lW1, lW2 = lW1.astype(bf), lW2.astype(bf)

    ones_d = jnp.ones((BD, DW), jnp.float32)
    zer_d = jnp.zeros((ROWS_PER_SUB, DW), jnp.float32)
    zer_f = jnp.zeros((ZR, CW), jnp.float32)

    deg2 = _deg_kernel(dstd, ones_d, zer_d)

    c0 = feat.shape[1] // CW
    hc = feat.reshape(N, c0, CW).transpose(1, 0, 2)

    segsum_in = _make_segsum(c0)
    segsum_h = _make_segsum(2048 // CW)

    agg = segsum_in(hc, srcw, dstw, zer_f)
    hc = _tc_layer(agg, deg2, W1, b1, g1, be1)
    for (W, b, g, be) in ((W2, b2, g2, be2), (W3, b3, g3, be3),
                          (W4, b4, g4, be4), (W5, b5, g5, be5)):
        agg = segsum_h(hc, srcw, dstw, zer_f)
        hc = _tc_layer(agg, deg2, W, b, g, be)
    return _tc_final(hc, lW1, lb1, g1, be1, lW2, lb2)
